# baseline (device time: 187261 ns/iter reference)
import jax
import jax.numpy as jnp
from jax import lax
from jax.experimental import pallas as pl
from jax.experimental.pallas import tpu as pltpu

N_DEV = 16
SQ = 1024
SKV_PER = 1024
HQ = 8
DH = 128
D = HQ * DH
BLK = 64
CH = SQ // N_DEV
QC = 256
SCALE = 0.08838834764831843
NEG = -1e9

_MESH = pl.DeviceIdType.MESH


def _attn_body(x_ref, wq_ref, k_ref, v_ref, ctx_ref, st_ref):
    my = lax.axis_index("i")
    qc0 = pl.program_id(0) * QC
    bf16 = jnp.bfloat16
    q_chunk = jnp.dot(x_ref[0].astype(bf16), wq_ref[...].astype(bf16),
                      preferred_element_type=jnp.float32)
    qb = (lax.broadcasted_iota(jnp.int32, (QC, SKV_PER), 0) + qc0) // BLK
    kb = (lax.broadcasted_iota(jnp.int32, (QC, SKV_PER), 1) // BLK
          + my * (SKV_PER // BLK))
    mask = (qb == kb) | (kb == 0) | ((qb + kb) % 3 == 0)
    for h in range(HQ):
        q_h = q_chunk[:, h * DH:(h + 1) * DH].astype(bf16)
        k_h = k_ref[0, :, h, :].astype(bf16)
        v_h = v_ref[0, :, h, :].astype(bf16)
        s = lax.dot_general(q_h, k_h, (((1,), (1,)), ((), ())),
                            preferred_element_type=jnp.float32) * SCALE
        s = jnp.where(mask, s, NEG)
        m_h = jnp.max(s, axis=1, keepdims=True)
        w = jnp.exp(s - m_h)
        l_h = jnp.sum(w, axis=1, keepdims=True)
        ctx_h = jnp.dot(w.astype(bf16), v_h,
                        preferred_element_type=jnp.float32)
        for c in range(QC // CH):
            rows = slice(c * CH, (c + 1) * CH)
            ctx_ref[c, :, h, :] = ctx_h[rows, :]
            st_ref[c, :, h:h + 1] = m_h[rows, :]
            st_ref[c, :, HQ + h:HQ + h + 1] = l_h[rows, :]


def _combine(ctx_a, st_a, ctx_r, st_r):
    m_a, l_a = st_a[:, :HQ], st_a[:, HQ:]
    m_r, l_r = st_r[:, :HQ], st_r[:, HQ:]
    m_n = jnp.maximum(m_a, m_r)
    ea = jnp.exp(m_a - m_n)
    er = jnp.exp(m_r - m_n)
    ctx_n = ctx_a * ea[:, :, None] + ctx_r * er[:, :, None]
    st_n = jnp.concatenate([m_n, l_a * ea + l_r * er], axis=1)
    return ctx_n, st_n


def _ring_body(ctx_ref, st_ref, wo_ref, out_ref,
               rs_ctx, rs_st, bc_buf,
               rs_ctx_send, rs_ctx_recv, rs_st_send, rs_st_recv,
               bc_send, bc_recv):
    my = lax.axis_index("i")

    barrier = pltpu.get_barrier_semaphore()

    def _peer(j):
        return j + jnp.where(j >= my, 1, 0)

    def bar_sig(j, c):
        pl.semaphore_signal(barrier, inc=1, device_id=(_peer(j),),
                            device_id_type=_MESH)
        return c
    lax.fori_loop(0, N_DEV - 1, bar_sig, 0)
    pl.semaphore_wait(barrier, N_DEV - 1)

    def rs_send(j, c):
        jj = _peer(j)
        r1 = pltpu.make_async_remote_copy(
            src_ref=ctx_ref.at[jj], dst_ref=rs_ctx.at[my],
            send_sem=rs_ctx_send.at[jj], recv_sem=rs_ctx_recv.at[my],
            device_id=(jj,), device_id_type=_MESH)
        r2 = pltpu.make_async_remote_copy(
            src_ref=st_ref.at[jj], dst_ref=rs_st.at[my],
            send_sem=rs_st_send.at[jj], recv_sem=rs_st_recv.at[my],
            device_id=(jj,), device_id_type=_MESH)
        r1.start()
        r2.start()
        return c
    lax.fori_loop(0, N_DEV - 1, rs_send, 0)

    def rs_recv(j, carry):
        ctx_a, st_a = carry
        jj = _peer(j)
        d1 = pltpu.make_async_remote_copy(
            src_ref=rs_ctx.at[jj], dst_ref=rs_ctx.at[jj],
            send_sem=rs_ctx_send.at[jj], recv_sem=rs_ctx_recv.at[jj],
            device_id=(my,), device_id_type=_MESH)
        d2 = pltpu.make_async_remote_copy(
            src_ref=rs_st.at[jj], dst_ref=rs_st.at[jj],
            send_sem=rs_st_send.at[jj], recv_sem=rs_st_recv.at[jj],
            device_id=(my,), device_id_type=_MESH)
        d1.wait_recv()
        d2.wait_recv()
        return _combine(ctx_a, st_a, rs_ctx[jj], rs_st[jj])

    ctx_f, st_f = lax.fori_loop(
        0, N_DEV - 1, rs_recv, (ctx_ref[my], st_ref[my]))

    inv_l = 1.0 / st_f[:, HQ:]
    ctx_n = ctx_f * inv_l[:, :, None]
    o = jnp.zeros((CH, D), dtype=jnp.float32)
    for h in range(HQ):
        o = o + jnp.dot(ctx_n[:, h, :].astype(jnp.bfloat16),
                        wo_ref[h * DH:(h + 1) * DH, :].astype(jnp.bfloat16),
                        preferred_element_type=jnp.float32)
    out_ref[0, pl.ds(my * CH, CH), :] = o
    bc_buf[my] = o

    def bc_push(j, c):
        jj = _peer(j)
        r = pltpu.make_async_remote_copy(
            src_ref=bc_buf.at[my], dst_ref=bc_buf.at[my],
            send_sem=bc_send.at[jj], recv_sem=bc_recv.at[my],
            device_id=(jj,), device_id_type=_MESH)
        r.start()
        return c
    lax.fori_loop(0, N_DEV - 1, bc_push, 0)

    def bc_pull(j, c):
        jj = _peer(j)
        d = pltpu.make_async_remote_copy(
            src_ref=bc_buf.at[jj], dst_ref=bc_buf.at[jj],
            send_sem=bc_send.at[jj], recv_sem=bc_recv.at[jj],
            device_id=(my,), device_id_type=_MESH)
        d.wait_recv()
        out_ref[0, pl.ds(jj * CH, CH), :] = bc_buf[jj]
        return c
    lax.fori_loop(0, N_DEV - 1, bc_pull, 0)

    def drain(j, c):
        jj = _peer(j)
        d1 = pltpu.make_async_remote_copy(
            src_ref=ctx_ref.at[jj], dst_ref=rs_ctx.at[jj],
            send_sem=rs_ctx_send.at[jj], recv_sem=rs_ctx_recv.at[jj],
            device_id=(my,), device_id_type=_MESH)
        d2 = pltpu.make_async_remote_copy(
            src_ref=st_ref.at[jj], dst_ref=rs_st.at[jj],
            send_sem=rs_st_send.at[jj], recv_sem=rs_st_recv.at[jj],
            device_id=(my,), device_id_type=_MESH)
        d3 = pltpu.make_async_remote_copy(
            src_ref=bc_buf.at[my], dst_ref=bc_buf.at[my],
            send_sem=bc_send.at[jj], recv_sem=bc_recv.at[jj],
            device_id=(my,), device_id_type=_MESH)
        d1.wait_send()
        d2.wait_send()
        d3.wait_send()
        return c
    lax.fori_loop(0, N_DEV - 1, drain, 0)


def kernel(x, Wq, K_ext, V_ext, Wo):
    ctx, st = pl.pallas_call(
        _attn_body,
        grid=(SQ // QC,),
        out_shape=[
            jax.ShapeDtypeStruct((N_DEV, CH, HQ, DH), jnp.float32),
            jax.ShapeDtypeStruct((N_DEV, CH, 2 * HQ), jnp.float32),
        ],
        in_specs=[
            pl.BlockSpec((1, QC, D), lambda i: (0, i, 0)),
            pl.BlockSpec((D, D), lambda i: (0, 0)),
            pl.BlockSpec((1, SKV_PER, HQ, DH), lambda i: (0, 0, 0, 0)),
            pl.BlockSpec((1, SKV_PER, HQ, DH), lambda i: (0, 0, 0, 0)),
        ],
        out_specs=[
            pl.BlockSpec((QC // CH, CH, HQ, DH), lambda i: (i, 0, 0, 0)),
            pl.BlockSpec((QC // CH, CH, 2 * HQ), lambda i: (i, 0, 0)),
        ],
        compiler_params=pltpu.CompilerParams(
            vmem_limit_bytes=63 * 1024 * 1024),
    )(x, Wq, K_ext, V_ext)

    return pl.pallas_call(
        _ring_body,
        out_shape=jax.ShapeDtypeStruct((1, SQ, D), jnp.float32),
        in_specs=[pl.BlockSpec(memory_space=pltpu.VMEM)] * 3,
        out_specs=pl.BlockSpec(memory_space=pltpu.VMEM),
        scratch_shapes=[
            pltpu.VMEM((N_DEV, CH, HQ, DH), jnp.float32),
            pltpu.VMEM((N_DEV, CH, 2 * HQ), jnp.float32),
            pltpu.VMEM((N_DEV, CH, D), jnp.float32),
            pltpu.SemaphoreType.DMA((N_DEV,)),
            pltpu.SemaphoreType.DMA((N_DEV,)),
            pltpu.SemaphoreType.DMA((N_DEV,)),
            pltpu.SemaphoreType.DMA((N_DEV,)),
            pltpu.SemaphoreType.DMA((N_DEV,)),
            pltpu.SemaphoreType.DMA((N_DEV,)),
        ],
        compiler_params=pltpu.CompilerParams(
            collective_id=0, vmem_limit_bytes=63 * 1024 * 1024),
    )(ctx, st, Wo)


# device time: 149401 ns/iter; 1.2534x vs baseline; 1.2534x over previous
import jax
import jax.numpy as jnp
from jax import lax
from jax.experimental import pallas as pl
from jax.experimental.pallas import tpu as pltpu

N_DEV = 16
SQ = 1024
SKV_PER = 1024
HQ = 8
DH = 128
D = HQ * DH
BLK = 64
CH = SQ // N_DEV
QC = 256
NSTEP = SQ // QC
SCALE = 0.08838834764831843
NEG = -1e9

_MESH = pl.DeviceIdType.MESH


def _combine(ctx_a, st_a, ctx_r, st_r):
    m_a, l_a = st_a[:, :HQ], st_a[:, HQ:]
    m_r, l_r = st_r[:, :HQ], st_r[:, HQ:]
    m_n = jnp.maximum(m_a, m_r)
    ea = jnp.exp(m_a - m_n)
    er = jnp.exp(m_r - m_n)
    ctx_n = ctx_a * ea[:, :, None] + ctx_r * er[:, :, None]
    st_n = jnp.concatenate([m_n, l_a * ea + l_r * er], axis=1)
    return ctx_n, st_n


def _body(x_ref, wq_ref, k_ref, v_ref, wo_ref, out_ref,
          loc_ctx, loc_st, rs_ctx, rs_st, bc_buf,
          rs_ctx_send, rs_ctx_recv, rs_st_send, rs_st_recv,
          bc_send, bc_recv):
    my = lax.axis_index("i")
    i = pl.program_id(0)
    barrier = pltpu.get_barrier_semaphore()

    def _peer(j):
        return j + jnp.where(j >= my, 1, 0)

    @pl.when(i == 0)
    def _():
        def bar_sig(j, c):
            pl.semaphore_signal(barrier, inc=1, device_id=(_peer(j),),
                                device_id_type=_MESH)
            return c
        lax.fori_loop(0, N_DEV - 1, bar_sig, 0)
        pl.semaphore_wait(barrier, N_DEV - 1)

    qc0 = i * QC
    q_chunk = jnp.dot(x_ref[0], wq_ref[...],
                      preferred_element_type=jnp.float32)
    qb = (lax.broadcasted_iota(jnp.int32, (QC, SKV_PER), 0) + qc0) // BLK
    kb = (lax.broadcasted_iota(jnp.int32, (QC, SKV_PER), 1) // BLK
          + my * (SKV_PER // BLK))
    mask = (qb == kb) | (kb == 0) | ((qb + kb) % 3 == 0)
    for h in range(HQ):
        q_h = q_chunk[:, h * DH:(h + 1) * DH]
        k_h = k_ref[0, :, h, :]
        v_h = v_ref[0, :, h, :]
        s = lax.dot_general(q_h, k_h, (((1,), (1,)), ((), ())),
                            preferred_element_type=jnp.float32) * SCALE
        s = jnp.where(mask, s, NEG)
        m_h = jnp.max(s, axis=1, keepdims=True)
        w = jnp.exp(s - m_h)
        l_h = jnp.sum(w, axis=1, keepdims=True)
        ctx_h = jnp.dot(w, v_h, preferred_element_type=jnp.float32)
        for c in range(QC // CH):
            rows = slice(c * CH, (c + 1) * CH)
            ck = i * (QC // CH) + c
            loc_ctx[ck, :, h, :] = ctx_h[rows, :]
            loc_st[ck, :, h:h + 1] = m_h[rows, :]
            loc_st[ck, :, HQ + h:HQ + h + 1] = l_h[rows, :]

    for c in range(QC // CH):
        ck = i * (QC // CH) + c

        @pl.when(ck != my)
        def _(ck=ck):
            r1 = pltpu.make_async_remote_copy(
                src_ref=loc_ctx.at[ck], dst_ref=rs_ctx.at[my],
                send_sem=rs_ctx_send.at[ck], recv_sem=rs_ctx_recv.at[my],
                device_id=(ck,), device_id_type=_MESH)
            r2 = pltpu.make_async_remote_copy(
                src_ref=loc_st.at[ck], dst_ref=rs_st.at[my],
                send_sem=rs_st_send.at[ck], recv_sem=rs_st_recv.at[my],
                device_id=(ck,), device_id_type=_MESH)
            r1.start()
            r2.start()

    @pl.when(i == NSTEP - 1)
    def _():
        def rs_recv(j, carry):
            ctx_a, st_a = carry
            jj = _peer(j)
            d1 = pltpu.make_async_remote_copy(
                src_ref=rs_ctx.at[jj], dst_ref=rs_ctx.at[jj],
                send_sem=rs_ctx_send.at[jj], recv_sem=rs_ctx_recv.at[jj],
                device_id=(my,), device_id_type=_MESH)
            d2 = pltpu.make_async_remote_copy(
                src_ref=rs_st.at[jj], dst_ref=rs_st.at[jj],
                send_sem=rs_st_send.at[jj], recv_sem=rs_st_recv.at[jj],
                device_id=(my,), device_id_type=_MESH)
            d1.wait_recv()
            d2.wait_recv()
            return _combine(ctx_a, st_a, rs_ctx[jj], rs_st[jj])

        ctx_f, st_f = lax.fori_loop(
            0, N_DEV - 1, rs_recv, (loc_ctx[my], loc_st[my]))

        inv_l = 1.0 / st_f[:, HQ:]
        ctx_n = ctx_f * inv_l[:, :, None]
        o = jnp.zeros((CH, D), dtype=jnp.float32)
        for h in range(HQ):
            o = o + jnp.dot(ctx_n[:, h, :], wo_ref[h * DH:(h + 1) * DH, :],
                            preferred_element_type=jnp.float32)
        out_ref[0, pl.ds(my * CH, CH), :] = o
        bc_buf[my] = o

        def bc_push(j, c):
            jj = _peer(j)
            r = pltpu.make_async_remote_copy(
                src_ref=bc_buf.at[my], dst_ref=bc_buf.at[my],
                send_sem=bc_send.at[jj], recv_sem=bc_recv.at[my],
                device_id=(jj,), device_id_type=_MESH)
            r.start()
            return c
        lax.fori_loop(0, N_DEV - 1, bc_push, 0)

        def bc_pull(j, c):
            jj = _peer(j)
            d = pltpu.make_async_remote_copy(
                src_ref=bc_buf.at[jj], dst_ref=bc_buf.at[jj],
                send_sem=bc_send.at[jj], recv_sem=bc_recv.at[jj],
                device_id=(my,), device_id_type=_MESH)
            d.wait_recv()
            out_ref[0, pl.ds(jj * CH, CH), :] = bc_buf[jj]
            return c
        lax.fori_loop(0, N_DEV - 1, bc_pull, 0)

        def drain(j, c):
            jj = _peer(j)
            d1 = pltpu.make_async_remote_copy(
                src_ref=loc_ctx.at[jj], dst_ref=rs_ctx.at[jj],
                send_sem=rs_ctx_send.at[jj], recv_sem=rs_ctx_recv.at[jj],
                device_id=(my,), device_id_type=_MESH)
            d2 = pltpu.make_async_remote_copy(
                src_ref=loc_st.at[jj], dst_ref=rs_st.at[jj],
                send_sem=rs_st_send.at[jj], recv_sem=rs_st_recv.at[jj],
                device_id=(my,), device_id_type=_MESH)
            d3 = pltpu.make_async_remote_copy(
                src_ref=bc_buf.at[my], dst_ref=bc_buf.at[my],
                send_sem=bc_send.at[jj], recv_sem=bc_recv.at[jj],
                device_id=(my,), device_id_type=_MESH)
            d1.wait_send()
            d2.wait_send()
            d3.wait_send()
            return c
        lax.fori_loop(0, N_DEV - 1, drain, 0)


def kernel(x, Wq, K_ext, V_ext, Wo):
    return pl.pallas_call(
        _body,
        grid=(NSTEP,),
        out_shape=jax.ShapeDtypeStruct((1, SQ, D), jnp.float32),
        in_specs=[
            pl.BlockSpec((1, QC, D), lambda i: (0, i, 0)),
            pl.BlockSpec((D, D), lambda i: (0, 0)),
            pl.BlockSpec((1, SKV_PER, HQ, DH), lambda i: (0, 0, 0, 0)),
            pl.BlockSpec((1, SKV_PER, HQ, DH), lambda i: (0, 0, 0, 0)),
            pl.BlockSpec((D, D), lambda i: (0, 0)),
        ],
        out_specs=pl.BlockSpec((1, SQ, D), lambda i: (0, 0, 0)),
        scratch_shapes=[
            pltpu.VMEM((N_DEV, CH, HQ, DH), jnp.float32),
            pltpu.VMEM((N_DEV, CH, 2 * HQ), jnp.float32),
            pltpu.VMEM((N_DEV, CH, HQ, DH), jnp.float32),
            pltpu.VMEM((N_DEV, CH, 2 * HQ), jnp.float32),
            pltpu.VMEM((N_DEV, CH, D), jnp.float32),
            pltpu.SemaphoreType.DMA((N_DEV,)),
            pltpu.SemaphoreType.DMA((N_DEV,)),
            pltpu.SemaphoreType.DMA((N_DEV,)),
            pltpu.SemaphoreType.DMA((N_DEV,)),
            pltpu.SemaphoreType.DMA((N_DEV,)),
            pltpu.SemaphoreType.DMA((N_DEV,)),
        ],
        compiler_params=pltpu.CompilerParams(
            collective_id=0, vmem_limit_bytes=63 * 1024 * 1024),
    )(x, Wq, K_ext, V_ext, Wo)
